# Initial kernel scaffold; baseline (speedup 1.0000x reference)
import jax
import jax.numpy as jnp
from jax.experimental import pallas as pl

_K = 8192
_D = 256
_BM = 256


def _vq_body(x_ref, cbt_ref, out_ref):
    x = x_ref[...]
    cbt = cbt_ref[...]
    mm = jnp.dot(x, cbt, preferred_element_type=jnp.float32)
    in_sqr = jnp.sum(x * x, axis=1, keepdims=True)
    cb_sqr = jnp.sum(cbt * cbt, axis=0, keepdims=True)
    dist = (cb_sqr + in_sqr) - 2.0 * mm
    minv = jnp.min(dist, axis=1, keepdims=True)
    iota = jax.lax.broadcasted_iota(jnp.int32, dist.shape, 1)
    idx = jnp.min(jnp.where(dist == minv, iota, _K), axis=1)
    out_ref[0, 0, :] = idx


def kernel(z_e, codebook):
    lead_shape = z_e.shape[:-1]
    x = z_e.reshape(-1, _D)
    m = x.shape[0]
    nblk = m // _BM
    cbt = codebook.T
    out = pl.pallas_call(
        _vq_body,
        grid=(nblk,),
        in_specs=[
            pl.BlockSpec((_BM, _D), lambda i: (i, 0)),
            pl.BlockSpec((_D, _K), lambda i: (0, 0)),
        ],
        out_specs=pl.BlockSpec((1, 1, _BM), lambda i: (i, 0, 0)),
        out_shape=jax.ShapeDtypeStruct((nblk, 1, _BM), jnp.int32),
    )(x, cbt)
    return out.reshape(lead_shape)


# speedup vs baseline: 1.0004x; 1.0004x over previous
import jax
import jax.numpy as jnp
from jax.experimental import pallas as pl

_K = 8192
_D = 256
_BM = 256


def _vq_body(x_ref, cbt_ref, out_ref):
    x = x_ref[...]
    cbt = cbt_ref[...]
    mm = jnp.dot(x.astype(jnp.bfloat16), cbt.astype(jnp.bfloat16),
                 preferred_element_type=jnp.float32)
    cb_sqr = jnp.sum(cbt * cbt, axis=0, keepdims=True)
    score = cb_sqr - 2.0 * mm
    minv = jnp.min(score, axis=1, keepdims=True)
    iota = jax.lax.broadcasted_iota(jnp.int32, score.shape, 1)
    idx = jnp.min(jnp.where(score == minv, iota, _K), axis=1)
    out_ref[0, 0, :] = idx


def kernel(z_e, codebook):
    lead_shape = z_e.shape[:-1]
    x = z_e.reshape(-1, _D)
    m = x.shape[0]
    nblk = m // _BM
    cbt = codebook.T
    out = pl.pallas_call(
        _vq_body,
        grid=(nblk,),
        in_specs=[
            pl.BlockSpec((_BM, _D), lambda i: (i, 0)),
            pl.BlockSpec((_D, _K), lambda i: (0, 0)),
        ],
        out_specs=pl.BlockSpec((1, 1, _BM), lambda i: (i, 0, 0)),
        out_shape=jax.ShapeDtypeStruct((nblk, 1, _BM), jnp.int32),
    )(x, cbt)
    return out.reshape(lead_shape)
